# baseline (speedup 1.0000x reference)
import functools

import jax
import jax.numpy as jnp
from jax import lax
from jax.experimental import pallas as pl
from jax.experimental.pallas import tpu as pltpu
from jax.experimental.pallas import tpu_sc as plsc

VOCAB = 100000
FEAT = 64
ENTITY = 128

_NC = 2
_NS = 16
_NW = _NC * _NS

_ROW_BLOCK = 5000
_CH = 128
_NBUF = 2
_SPB = 2


def _proj_body(table_ref, w_ref, b_ref, out_ref):
    out_ref[...] = (
        jnp.dot(table_ref[...], w_ref[...], preferred_element_type=jnp.float32)
        + b_ref[...]
    )


def _project_table(table, W, b):
    return pl.pallas_call(
        _proj_body,
        grid=(VOCAB // _ROW_BLOCK,),
        in_specs=[
            pl.BlockSpec((_ROW_BLOCK, FEAT), lambda i: (i, 0)),
            pl.BlockSpec((FEAT, ENTITY), lambda i: (0, 0)),
            pl.BlockSpec((1, ENTITY), lambda i: (0, 0)),
        ],
        out_specs=pl.BlockSpec((_ROW_BLOCK, ENTITY), lambda i: (i, 0)),
        out_shape=jax.ShapeDtypeStruct((VOCAB, ENTITY), jnp.float32),
    )(table, W, b.reshape(1, ENTITY))


@functools.lru_cache(maxsize=None)
def _make_gather(total, bsz):
    n_per_w = total // _NW
    n_chunks = n_per_w // _CH
    n_super = n_chunks // _SPB
    n_groups = n_super // _NBUF
    mesh = plsc.VectorSubcoreMesh(core_axis_name="c", subcore_axis_name="s")

    scratch = [pltpu.VMEM((n_chunks, _CH), jnp.int32)]
    scratch += [pltpu.VMEM((_SPB * _CH, ENTITY), jnp.float32) for _ in range(_NBUF)]
    scratch += [pltpu.SemaphoreType.DMA for _ in range(2 * _NBUF)]

    @functools.partial(
        pl.kernel,
        mesh=mesh,
        out_type=jax.ShapeDtypeStruct((total // bsz, bsz, ENTITY), jnp.float32),
        scratch_types=scratch,
    )
    def gather_k(idx_hbm, proj_hbm, out_hbm, idx_v, *bufs):
        rows = bufs[:_NBUF]
        gsem = bufs[_NBUF:2 * _NBUF]
        osem = bufs[2 * _NBUF:]
        wid = lax.axis_index("s") * _NC + lax.axis_index("c")
        base = wid * n_per_w
        out_flat = out_hbm.reshape(total, ENTITY)
        pltpu.sync_copy(idx_hbm.at[wid], idx_v)

        def out_slice(s):
            return out_flat.at[pl.ds(base + s * _SPB * _CH, _SPB * _CH)]

        def group(g, carry):
            s0 = g * _NBUF
            handles = []
            for t in range(_NBUF):
                @pl.when(g > 0)
                def _():
                    pltpu.make_async_copy(
                        rows[t], out_slice(s0 - _NBUF + t), osem[t]
                    ).wait()
                handles.append([
                    pltpu.async_copy(
                        proj_hbm.at[idx_v.at[(s0 + t) * _SPB + r]],
                        rows[t].at[pl.ds(r * _CH, _CH)],
                        gsem[t],
                    )
                    for r in range(_SPB)
                ])
            for t in range(_NBUF):
                for h in handles[t]:
                    h.wait()
                pltpu.async_copy(rows[t], out_slice(s0 + t), osem[t])
            return carry

        lax.fori_loop(0, n_groups, group, 0)
        for t in range(_NBUF):
            pltpu.make_async_copy(
                rows[t], out_slice((n_groups - 1) * _NBUF + t), osem[t]
            ).wait()

    return gather_k


def kernel(x, table, W, b):
    bsz, seq = x.shape
    total = bsz * seq
    proj = _project_table(table, W, b)
    idx = jnp.transpose(x).reshape(_NW, total // _NW // _CH, _CH)
    out_t = _make_gather(total, bsz)(idx, proj)
    return jnp.transpose(out_t, (1, 0, 2))


# speedup vs baseline: 1.0158x; 1.0158x over previous
import functools

import jax
import jax.numpy as jnp
from jax import lax
from jax.experimental import pallas as pl
from jax.experimental.pallas import tpu as pltpu
from jax.experimental.pallas import tpu_sc as plsc

VOCAB = 100000
FEAT = 64
ENTITY = 128

_NC = 2
_NS = 16
_NW = _NC * _NS

_ROW_BLOCK = 10000
_CH = 128
_NBUF = 2
_SPB = 2


def _proj_body(table_ref, w_ref, b_ref, out_ref):
    out_ref[...] = (
        jnp.dot(table_ref[...], w_ref[...], preferred_element_type=jnp.float32)
        + b_ref[...]
    )


def _project_table(table, W, b):
    return pl.pallas_call(
        _proj_body,
        grid=(VOCAB // _ROW_BLOCK,),
        in_specs=[
            pl.BlockSpec((_ROW_BLOCK, FEAT), lambda i: (i, 0)),
            pl.BlockSpec((FEAT, ENTITY), lambda i: (0, 0)),
            pl.BlockSpec((1, ENTITY), lambda i: (0, 0)),
        ],
        out_specs=pl.BlockSpec((_ROW_BLOCK, ENTITY), lambda i: (i, 0)),
        out_shape=jax.ShapeDtypeStruct((VOCAB, ENTITY), jnp.float32),
    )(table, W, b.reshape(1, ENTITY))


@functools.lru_cache(maxsize=None)
def _make_gather(total, bsz):
    n_per_w = total // _NW
    n_chunks = n_per_w // _CH
    n_super = n_chunks // _SPB
    n_groups = n_super // _NBUF
    mesh = plsc.VectorSubcoreMesh(core_axis_name="c", subcore_axis_name="s")

    scratch = [pltpu.VMEM((n_chunks, _CH), jnp.int32)]
    scratch += [pltpu.VMEM((_SPB * _CH, ENTITY), jnp.float32) for _ in range(_NBUF)]
    scratch += [pltpu.SemaphoreType.DMA for _ in range(2 * _NBUF)]

    @functools.partial(
        pl.kernel,
        mesh=mesh,
        out_type=jax.ShapeDtypeStruct((total // bsz, bsz, ENTITY), jnp.float32),
        scratch_types=scratch,
    )
    def gather_k(idx_hbm, proj_hbm, out_hbm, idx_v, *bufs):
        rows = bufs[:_NBUF]
        gsem = bufs[_NBUF:2 * _NBUF]
        osem = bufs[2 * _NBUF:]
        wid = lax.axis_index("s") * _NC + lax.axis_index("c")
        base = wid * n_per_w
        out_flat = out_hbm.reshape(total, ENTITY)
        pltpu.sync_copy(idx_hbm.at[wid], idx_v)

        def out_slice(s):
            return out_flat.at[pl.ds(base + s * _SPB * _CH, _SPB * _CH)]

        def group(g, carry):
            s0 = g * _NBUF
            handles = []
            for t in range(_NBUF):
                @pl.when(g > 0)
                def _():
                    pltpu.make_async_copy(
                        rows[t], out_slice(s0 - _NBUF + t), osem[t]
                    ).wait()
                handles.append([
                    pltpu.async_copy(
                        proj_hbm.at[idx_v.at[(s0 + t) * _SPB + r]],
                        rows[t].at[pl.ds(r * _CH, _CH)],
                        gsem[t],
                    )
                    for r in range(_SPB)
                ])
            for t in range(_NBUF):
                for h in handles[t]:
                    h.wait()
                pltpu.async_copy(rows[t], out_slice(s0 + t), osem[t])
            return carry

        lax.fori_loop(0, n_groups, group, 0)
        for t in range(_NBUF):
            pltpu.make_async_copy(
                rows[t], out_slice((n_groups - 1) * _NBUF + t), osem[t]
            ).wait()

    return gather_k


def kernel(x, table, W, b):
    bsz, seq = x.shape
    total = bsz * seq
    proj = _project_table(table, W, b)
    idx = jnp.transpose(x).reshape(_NW, total // _NW // _CH, _CH)
    out_t = _make_gather(total, bsz)(idx, proj)
    return jnp.transpose(out_t, (1, 0, 2))
